# baseline (speedup 1.0000x reference)
import functools

import jax
import jax.numpy as jnp
from jax import lax
from jax.experimental import pallas as pl
from jax.experimental.pallas import tpu as pltpu
from jax.experimental.pallas import tpu_sc as plsc

N = 10000
E = 320000
D = 128
OUT = 3
G = 64

NC, NS, L = 2, 16, 16
NW = NC * NS

NP = 10240
EW = 10240
EP = NW * EW
CH = 128
NCHUNK_DEG = EW // CH
CHE = 64
NCHUNK = EW // CHE
NBUF = 4
RPS = NP // NS

_mesh = plsc.VectorSubcoreMesh(core_axis_name="c", subcore_axis_name="s")
_f32 = jnp.float32


@functools.partial(
    pl.kernel,
    out_type=jax.ShapeDtypeStruct((NW, NP), _f32),
    mesh=_mesh,
    scratch_types=[pltpu.VMEM((NP,), _f32),
                   pltpu.VMEM((EW,), jnp.int32)],
    compiler_params=pltpu.CompilerParams(needs_layout_passes=False),
)
def _deg(dst_hbm, zeros1_hbm, ecnt_hbm, cnt_v, idx_v):
    c = lax.axis_index("c")
    s = lax.axis_index("s")
    wid = s * NC + c
    pltpu.sync_copy(zeros1_hbm, cnt_v)
    pltpu.sync_copy(dst_hbm.at[wid], idx_v)
    ones = jnp.full((L,), 1.0, _f32)

    def chunk(i, carry):
        for k in range(CH // L):
            plsc.addupdate_scatter(
                cnt_v, [idx_v[pl.ds(i * CH + k * L, L)]], ones)
        return carry

    lax.fori_loop(0, NCHUNK_DEG, chunk, 0)
    pltpu.sync_copy(cnt_v, ecnt_hbm.at[wid])


@functools.partial(
    pl.kernel,
    out_type=jax.ShapeDtypeStruct((NC, NP, D), _f32),
    mesh=_mesh,
    scratch_types=[pltpu.VMEM((EW,), jnp.int32)]
                  + [pltpu.VMEM((1, CHE), jnp.int32)] * NBUF
                  + [pltpu.VMEM((CHE, D), _f32)] * NBUF
                  + [pltpu.VMEM_SHARED((NP, D), _f32)]
                  + [pltpu.SemaphoreType.DMA] * (2 * NBUF),
)
def _edge_scatter(u_hbm, src_hbm, dst_hbm, zeros2_hbm, out_hbm,
                  si_v, *rest):
    di_v = rest[:NBUF]
    rows_v = rest[NBUF:2 * NBUF]
    acc_sh = rest[2 * NBUF]
    gsem = rest[2 * NBUF + 1:2 * NBUF + 1 + NBUF]
    dsem = rest[2 * NBUF + 1 + NBUF:]
    c = lax.axis_index("c")
    s = lax.axis_index("s")
    wid = s * NC + c
    rs = s * RPS
    pltpu.sync_copy(src_hbm.at[wid], si_v)
    pltpu.sync_copy(zeros2_hbm.at[pl.ds(rs, RPS)], acc_sh.at[pl.ds(rs, RPS)])
    plsc.subcore_barrier()

    def sidx(c):
        return si_v.at[pl.ds(c * CHE, CHE)]

    for k in range(NBUF):
        pltpu.async_copy(dst_hbm.at[wid, k], di_v[k], dsem[k])
        pltpu.async_copy(u_hbm.at[sidx(k)], rows_v[k], gsem[k])

    def quad(i, carry):
        c0 = NBUF * i
        for k in range(NBUF):
            pltpu.make_async_copy(u_hbm.at[sidx(c0 + k)], rows_v[k],
                                  gsem[k]).wait()
            pltpu.make_async_copy(dst_hbm.at[wid, 0], di_v[k],
                                  dsem[k]).wait()
            pltpu.sync_copy(rows_v[k], acc_sh.at[di_v[k].at[0]], add=True)

            @pl.when(i < NCHUNK // NBUF - 1)
            def _():
                pltpu.async_copy(dst_hbm.at[wid, c0 + k + NBUF], di_v[k],
                                 dsem[k])
                pltpu.async_copy(u_hbm.at[sidx(c0 + k + NBUF)], rows_v[k],
                                 gsem[k])

        return carry

    lax.fori_loop(0, NCHUNK // NBUF, quad, 0)
    plsc.subcore_barrier()
    pltpu.sync_copy(acc_sh.at[pl.ds(rs, RPS)], out_hbm.at[c, pl.ds(rs, RPS)])


R = 2560
NBLK = NP // R


def _prep1_body(x_ref, cnt_ref, w_ref, u_ref, dinv_ref):
    i = pl.program_id(0)
    dn = (((0,), (0,)), ((), ()))
    tot = lax.dot_general(cnt_ref[...], jnp.ones((NW, 1), _f32), dn,
                          preferred_element_type=_f32)
    rows = i * R + lax.broadcasted_iota(jnp.int32, (R, 1), 0)
    dinv = jnp.where(rows < N, lax.rsqrt(tot + 1.0), 0.0)
    u_ref[...] = jnp.where(
        rows < N,
        jnp.dot(x_ref[...], w_ref[...], preferred_element_type=_f32) * dinv,
        0.0)
    dinv_ref[...] = dinv


def _prep1(x, ecnt, W1):
    return pl.pallas_call(
        _prep1_body,
        grid=(NBLK,),
        in_specs=[pl.BlockSpec((R, D), lambda i: (i, 0)),
                  pl.BlockSpec((NW, R), lambda i: (0, i)),
                  pl.BlockSpec((D, D), lambda i: (0, 0))],
        out_specs=[pl.BlockSpec((R, D), lambda i: (i, 0)),
                   pl.BlockSpec((R, 1), lambda i: (i, 0))],
        out_shape=[jax.ShapeDtypeStruct((NP, D), _f32),
                   jax.ShapeDtypeStruct((NP, 1), _f32)],
    )(x, ecnt, W1)


def _mid_body(y_ref, u_ref, dinv_ref, b_ref, w_ref, o_ref):
    ys = jnp.sum(y_ref[...], axis=0)
    dinv = dinv_ref[...]
    h = jnp.maximum((ys + u_ref[...]) * dinv + b_ref[...], 0.0)
    o_ref[...] = jnp.dot(h, w_ref[...], preferred_element_type=_f32) * dinv


def _mid(y, u, dinv, b, W):
    return pl.pallas_call(
        _mid_body,
        grid=(NBLK,),
        in_specs=[pl.BlockSpec((NC, R, D), lambda i: (0, i, 0)),
                  pl.BlockSpec((R, D), lambda i: (i, 0)),
                  pl.BlockSpec((R, 1), lambda i: (i, 0)),
                  pl.BlockSpec((1, D), lambda i: (0, 0)),
                  pl.BlockSpec((D, D), lambda i: (0, 0))],
        out_specs=pl.BlockSpec((R, D), lambda i: (i, 0)),
        out_shape=jax.ShapeDtypeStruct((NP, D), _f32),
    )(y, u, dinv, b, W)


def _tail_body(y_ref, u_ref, dinv_ref, b_ref, batch_ref, wc_ref, bc_ref,
               o_ref, sums_ref, cnt_ref):
    i = pl.program_id(0)
    ys = jnp.sum(y_ref[...], axis=0)
    h = jnp.maximum((ys + u_ref[...]) * dinv_ref[...] + b_ref[...], 0.0)
    cols = lax.broadcasted_iota(jnp.int32, (R, G), 1)
    rows = i * R + lax.broadcasted_iota(jnp.int32, (R, 1), 0)
    oh = ((batch_ref[...] == cols) & (rows < N)).astype(_f32)
    dn = (((0,), (0,)), ((), ()))

    @pl.when(i == 0)
    def _():
        sums_ref[...] = jnp.zeros_like(sums_ref)
        cnt_ref[...] = jnp.zeros_like(cnt_ref)

    sums_ref[...] += lax.dot_general(oh, h, dn, preferred_element_type=_f32)
    cnt_ref[...] += lax.dot_general(oh, jnp.ones((R, 1), _f32), dn,
                                    preferred_element_type=_f32)

    @pl.when(i == NBLK - 1)
    def _():
        pooled = sums_ref[...] / jnp.maximum(cnt_ref[...], 1.0)
        o_ref[...] = jnp.dot(pooled, wc_ref[...],
                             preferred_element_type=_f32) + bc_ref[...]


def _tail(y, u, dinv, b, batch_col, Wc, bc):
    return pl.pallas_call(
        _tail_body,
        grid=(NBLK,),
        in_specs=[pl.BlockSpec((NC, R, D), lambda i: (0, i, 0)),
                  pl.BlockSpec((R, D), lambda i: (i, 0)),
                  pl.BlockSpec((R, 1), lambda i: (i, 0)),
                  pl.BlockSpec((1, D), lambda i: (0, 0)),
                  pl.BlockSpec((R, 1), lambda i: (i, 0)),
                  pl.BlockSpec((D, OUT), lambda i: (0, 0)),
                  pl.BlockSpec((1, OUT), lambda i: (0, 0))],
        out_specs=pl.BlockSpec((G, OUT), lambda i: (0, 0)),
        out_shape=jax.ShapeDtypeStruct((G, OUT), _f32),
        scratch_shapes=[pltpu.VMEM((G, D), _f32),
                        pltpu.VMEM((G, 1), _f32)],
    )(y, u, dinv, b, batch_col, Wc, bc)


def kernel(x, edge_index, batch, W1, b1, W2, b2, W3, b3, Wc, bc):
    src = edge_index[0]
    dst = edge_index[1]
    pad_idx = N + (jnp.arange(EP - E, dtype=jnp.int32) % (NP - N))
    src_p = jnp.concatenate([src, pad_idx])
    dst_p = jnp.concatenate([dst, pad_idx])
    src2 = src_p.reshape(NW, EW)
    dst2 = dst_p.reshape(NW, EW)
    dst4 = dst_p.reshape(NW, NCHUNK, 1, CHE)
    zeros1 = jnp.zeros((NP,), _f32)
    zeros2 = jnp.zeros((NP, D), _f32)

    ecnt = _deg(dst2, zeros1)
    u1, dinv = _prep1(x, ecnt, W1)
    y1 = _edge_scatter(u1, src2, dst4, zeros2)
    u2 = _mid(y1, u1, dinv, b1.reshape(1, D), W2)
    y2 = _edge_scatter(u2, src2, dst4, zeros2)
    u3 = _mid(y2, u2, dinv, b2.reshape(1, D), W3)
    y3 = _edge_scatter(u3, src2, dst4, zeros2)
    return _tail(y3, u3, dinv, b3.reshape(1, D), batch.reshape(N, 1),
                 Wc, bc.reshape(1, OUT))
